# Initial kernel scaffold; baseline (speedup 1.0000x reference)
import jax
import jax.numpy as jnp
from jax.experimental import pallas as pl


def kernel(x, w1, w2, conv_w, conv_b):
    raise NotImplementedError("write your pallas kernel here")


# speedup vs baseline: 11.4931x; 11.4931x over previous
import jax
import jax.numpy as jnp
from jax import lax
from jax.experimental import pallas as pl

K_NN = 32


def _body(xf_ref, w1_ref, w2_ref, out_ref):
    N = xf_ref.shape[1]
    DC = xf_ref.shape[2]
    C = w1_ref.shape[0]
    D = DC // C

    xf = xf_ref[0]

    w1 = w1_ref[...]
    w1n = w1 / jnp.maximum(
        jnp.sqrt(jnp.sum(w1 * w1, axis=1, keepdims=True)), 1e-12)
    wt = jnp.concatenate([w1n.T] * D, axis=1)
    w2 = w2_ref[...]
    w2n = w2 / jnp.maximum(
        jnp.sqrt(jnp.sum(w2 * w2, axis=0, keepdims=True)), 1e-12)

    sq = jnp.sum(xf * xf, axis=1, keepdims=True)
    inner = lax.dot_general(xf, xf, (((1,), (1,)), ((), ())),
                            preferred_element_type=jnp.float32)
    adj = sq - 2.0 * inner + sq.T

    iota = lax.broadcasted_iota(jnp.int32, (N, N), 1)
    big = jnp.int32(1 << 30)
    kiota = lax.broadcasted_iota(jnp.int32, (K_NN, DC), 0)

    def step(k, carry):
        adj, acc = carry
        rowmin = jnp.min(adj, axis=1, keepdims=True)
        tied = adj == rowmin
        idxm = jnp.min(jnp.where(tied, iota, big), axis=1, keepdims=True)
        onehot = iota == idxm
        g = lax.dot_general(onehot.astype(jnp.float32), xf,
                            (((1,), (0,)), ((), ())),
                            preferred_element_type=jnp.float32)
        wk = jnp.sum(jnp.where(kiota == k, wt, 0.0), axis=0, keepdims=True)
        acc = acc + g * wk
        adj = jnp.where(onehot, jnp.float32(jnp.inf), adj)
        return adj, acc

    acc0 = jnp.zeros((N, DC), dtype=jnp.float32)
    _, acc = lax.fori_loop(0, K_NN, step, (adj, acc0))

    pieces = []
    for d in range(D):
        pieces.append(lax.dot_general(acc[:, d * C:(d + 1) * C], w2n,
                                      (((1,), (0,)), ((), ())),
                                      preferred_element_type=jnp.float32))
    out_ref[0] = jnp.concatenate(pieces, axis=1)


def kernel(x, w1, w2, conv_w, conv_b):
    B, N, D, C = x.shape
    O = w2.shape[1]
    xf = x.reshape(B, N, D * C)
    out = pl.pallas_call(
        _body,
        grid=(B,),
        in_specs=[
            pl.BlockSpec((1, N, D * C), lambda b: (b, 0, 0)),
            pl.BlockSpec((C, K_NN), lambda b: (0, 0)),
            pl.BlockSpec((C, O), lambda b: (0, 0)),
        ],
        out_specs=pl.BlockSpec((1, N, D * O), lambda b: (b, 0, 0)),
        out_shape=jax.ShapeDtypeStruct((B, N, D * O), jnp.float32),
    )(xf, w1, w2)
    return out.reshape(B, N, D, O)


# speedup vs baseline: 11.5753x; 1.0072x over previous
import jax
import jax.numpy as jnp
from jax import lax
from jax.experimental import pallas as pl

K_NN = 32


def _body(xf_ref, w1_ref, w2_ref, out_ref):
    N = xf_ref.shape[1]
    DC = xf_ref.shape[2]
    C = w1_ref.shape[0]
    D = DC // C

    xf = xf_ref[0]

    w1 = w1_ref[...]
    w1n = w1 / jnp.maximum(
        jnp.sqrt(jnp.sum(w1 * w1, axis=1, keepdims=True)), 1e-12)
    wt = jnp.concatenate([w1n.T] * D, axis=1)
    w2 = w2_ref[...]
    w2n = w2 / jnp.maximum(
        jnp.sqrt(jnp.sum(w2 * w2, axis=0, keepdims=True)), 1e-12)

    sq = jnp.sum(xf * xf, axis=1, keepdims=True)
    inner = lax.dot_general(xf, xf, (((1,), (1,)), ((), ())),
                            preferred_element_type=jnp.float32)
    adj = sq - 2.0 * inner + sq.T

    iota = lax.broadcasted_iota(jnp.int32, (N, N), 1)
    big = jnp.int32(1 << 30)
    kiota = lax.broadcasted_iota(jnp.int32, (K_NN, DC), 0)
    xf_bf = xf.astype(jnp.bfloat16)

    def step(k, carry):
        adj, acc = carry
        rowmin = jnp.min(adj, axis=1, keepdims=True)
        tied = adj == rowmin
        idxm = jnp.min(jnp.where(tied, iota, big), axis=1, keepdims=True)
        onehot = iota == idxm
        g = lax.dot_general(onehot.astype(jnp.bfloat16), xf_bf,
                            (((1,), (0,)), ((), ())),
                            preferred_element_type=jnp.float32)
        wk = jnp.sum(jnp.where(kiota == k, wt, 0.0), axis=0, keepdims=True)
        acc = acc + g * wk
        adj = jnp.where(onehot, jnp.float32(jnp.inf), adj)
        return adj, acc

    acc0 = jnp.zeros((N, DC), dtype=jnp.float32)
    _, acc = lax.fori_loop(0, K_NN, step, (adj, acc0))

    pieces = []
    for d in range(D):
        pieces.append(lax.dot_general(acc[:, d * C:(d + 1) * C], w2n,
                                      (((1,), (0,)), ((), ())),
                                      preferred_element_type=jnp.float32))
    out_ref[0] = jnp.concatenate(pieces, axis=1)


def kernel(x, w1, w2, conv_w, conv_b):
    B, N, D, C = x.shape
    O = w2.shape[1]
    xf = x.reshape(B, N, D * C)
    out = pl.pallas_call(
        _body,
        grid=(B,),
        in_specs=[
            pl.BlockSpec((1, N, D * C), lambda b: (b, 0, 0)),
            pl.BlockSpec((C, K_NN), lambda b: (0, 0)),
            pl.BlockSpec((C, O), lambda b: (0, 0)),
        ],
        out_specs=pl.BlockSpec((1, N, D * O), lambda b: (b, 0, 0)),
        out_shape=jax.ShapeDtypeStruct((B, N, D * O), jnp.float32),
    )(xf, w1, w2)
    return out.reshape(B, N, D, O)
